# baseline (speedup 1.0000x reference)
from functools import partial

import jax
import jax.numpy as jnp
from jax.experimental import pallas as pl
from jax.experimental.pallas import tpu as pltpu

_NCORES = 1


def _enc_kernel(x_ref, wih_ref, whh_ref, bih_ref, bhh_ref, ua_ref,
                states_ref, projs_ref, hfinal_ref, h_scr):
    t = pl.program_id(1)
    Hp = h_scr.shape[1]

    @pl.when(t == 0)
    def _():
        h_scr[...] = jnp.zeros_like(h_scr)

    x = x_ref[0]
    h = h_scr[...]

    gx = jnp.dot(x, wih_ref[...], preferred_element_type=jnp.float32) + bih_ref[...]
    gh = jnp.dot(h, whh_ref[...], preferred_element_type=jnp.float32) + bhh_ref[...]

    r = jax.nn.sigmoid(gx[:, :Hp] + gh[:, :Hp])
    z = jax.nn.sigmoid(gx[:, Hp:2 * Hp] + gh[:, Hp:2 * Hp])
    n = jnp.tanh(gx[:, 2 * Hp:] + r * gh[:, 2 * Hp:])
    h_new = (1.0 - z) * n + z * h

    h_scr[...] = h_new
    states_ref[0] = h_new
    projs_ref[0] = jnp.dot(h_new, ua_ref[...], preferred_element_type=jnp.float32)

    @pl.when(t == pl.num_programs(1) - 1)
    def _():
        hfinal_ref[...] = h_new


def _run_encoder(emb_src, enc_wih, enc_whh, enc_bih, enc_bhh, dec_ua):
    T, B, Ep = emb_src.shape
    Hp = enc_whh.shape[0]
    Bblk = B // _NCORES
    states, projs, h_final = pl.pallas_call(
        _enc_kernel,
        out_shape=(jax.ShapeDtypeStruct((T, B, Hp), jnp.float32),
                   jax.ShapeDtypeStruct((T, B, Hp), jnp.float32),
                   jax.ShapeDtypeStruct((B, Hp), jnp.float32)),
        grid_spec=pltpu.PrefetchScalarGridSpec(
            num_scalar_prefetch=0,
            grid=(_NCORES, T),
            in_specs=[
                pl.BlockSpec((1, Bblk, Ep), lambda b, t: (t, b, 0)),
                pl.BlockSpec((Ep, 3 * Hp), lambda b, t: (0, 0)),
                pl.BlockSpec((Hp, 3 * Hp), lambda b, t: (0, 0)),
                pl.BlockSpec((1, 3 * Hp), lambda b, t: (0, 0)),
                pl.BlockSpec((1, 3 * Hp), lambda b, t: (0, 0)),
                pl.BlockSpec((Hp, Hp), lambda b, t: (0, 0)),
            ],
            out_specs=[
                pl.BlockSpec((1, Bblk, Hp), lambda b, t: (t, b, 0)),
                pl.BlockSpec((1, Bblk, Hp), lambda b, t: (t, b, 0)),
                pl.BlockSpec((Bblk, Hp), lambda b, t: (b, 0)),
            ],
            scratch_shapes=[pltpu.VMEM((Bblk, Hp), jnp.float32)],
        ),
        compiler_params=pltpu.CompilerParams(
            dimension_semantics=("parallel", "arbitrary")),
    )(emb_src, enc_wih, enc_whh, enc_bih, enc_bhh, dec_ua)
    return states, projs, h_final


def _dec_kernel(use_ref,
                teach_ref, enc_ref, projs_ref, emb_tab_ref,
                wa_ref, va_ref, win_ref, whh_ref, bih_ref, bhh_ref,
                wout_ref, bout_ref, hinit_ref,
                hcat_ref, h_scr, oh_scr, *, vocab):
    g = pl.program_id(1)
    B, Hp = h_scr.shape
    Vp = oh_scr.shape[1]
    nu = use_ref[1, g]

    @pl.when(g == 0)
    def _():
        h_scr[...] = hinit_ref[...]
        oh_scr[...] = jnp.zeros_like(oh_scr)

    h = h_scr[...]

    w_tf = (use_ref[0, g] > 0).astype(jnp.float32)
    inp_oh = w_tf * teach_ref[0] + (1.0 - w_tf) * oh_scr[...]
    emb = jnp.dot(inp_oh, emb_tab_ref[...], preferred_element_type=jnp.float32)

    proj_h = jnp.dot(h, wa_ref[...], preferred_element_type=jnp.float32)
    energy = jnp.tanh(projs_ref[...] + proj_h[None, :, :])
    scores = jnp.sum(energy * va_ref[...][None, :, :], axis=-1)
    scores = scores - jnp.max(scores, axis=0, keepdims=True)
    expo = jnp.exp(scores)
    alpha = expo * pl.reciprocal(jnp.sum(expo, axis=0, keepdims=True), approx=True)
    context = jnp.sum(alpha[:, :, None] * enc_ref[...], axis=0)

    xcat = jnp.concatenate([emb, context], axis=-1)
    gx = jnp.dot(xcat, win_ref[...], preferred_element_type=jnp.float32) + bih_ref[...]
    gh = jnp.dot(h, whh_ref[...], preferred_element_type=jnp.float32) + bhh_ref[...]
    r = jax.nn.sigmoid(gx[:, :Hp] + gh[:, :Hp])
    z = jax.nn.sigmoid(gx[:, Hp:2 * Hp] + gh[:, Hp:2 * Hp])
    n = jnp.tanh(gx[:, 2 * Hp:] + r * gh[:, 2 * Hp:])
    h_new = (1.0 - z) * n + z * h

    hcat = jnp.concatenate([h_new, context], axis=-1)
    hcat_ref[0] = hcat

    @pl.when(nu == 0)
    def _():
        logits = jnp.dot(hcat, wout_ref[...], preferred_element_type=jnp.float32) + bout_ref[...]
        v_iota = jax.lax.broadcasted_iota(jnp.int32, (B, Vp), 1).astype(jnp.float32)
        masked = jnp.where(v_iota < float(vocab), logits, -1e30)
        row_max = jnp.max(masked, axis=-1, keepdims=True)
        cand = jnp.where(masked == row_max, v_iota, float(Vp))
        first_idx = jnp.min(cand, axis=-1, keepdims=True)
        oh_scr[...] = (v_iota == first_idx).astype(jnp.float32)

    h_scr[...] = h_new


def _run_decoder(use2, teach_oh, enc_states, enc_proj, h_init,
                 emb_tab, wa, va, win, whh, bih, bhh, wout, bout, *, vocab):
    n_steps, B, Vp = teach_oh.shape
    T = enc_states.shape[0]
    Hp = h_init.shape[1]
    Ep = emb_tab.shape[1]
    Bblk = B // _NCORES
    kern = partial(_dec_kernel, vocab=vocab)
    hcat = pl.pallas_call(
        kern,
        out_shape=jax.ShapeDtypeStruct((n_steps, B, 2 * Hp), jnp.float32),
        grid_spec=pltpu.PrefetchScalarGridSpec(
            num_scalar_prefetch=1,
            grid=(_NCORES, n_steps),
            in_specs=[
                pl.BlockSpec((1, Bblk, Vp), lambda b, g, u: (g, b, 0)),
                pl.BlockSpec((T, Bblk, Hp), lambda b, g, u: (0, b, 0)),
                pl.BlockSpec((T, Bblk, Hp), lambda b, g, u: (0, b, 0)),
                pl.BlockSpec((Vp, Ep), lambda b, g, u: (0, 0)),
                pl.BlockSpec((Hp, Hp), lambda b, g, u: (0, 0)),
                pl.BlockSpec((1, Hp), lambda b, g, u: (0, 0)),
                pl.BlockSpec((Ep + Hp, 3 * Hp), lambda b, g, u: (0, 0)),
                pl.BlockSpec((Hp, 3 * Hp), lambda b, g, u: (0, 0)),
                pl.BlockSpec((1, 3 * Hp), lambda b, g, u: (0, 0)),
                pl.BlockSpec((1, 3 * Hp), lambda b, g, u: (0, 0)),
                pl.BlockSpec((2 * Hp, Vp), lambda b, g, u: (0, 0)),
                pl.BlockSpec((1, Vp), lambda b, g, u: (0, 0)),
                pl.BlockSpec((Bblk, Hp), lambda b, g, u: (b, 0)),
            ],
            out_specs=pl.BlockSpec((1, Bblk, 2 * Hp), lambda b, g, u: (g, b, 0)),
            scratch_shapes=[pltpu.VMEM((Bblk, Hp), jnp.float32),
                            pltpu.VMEM((Bblk, Vp), jnp.float32)],
        ),
        compiler_params=pltpu.CompilerParams(
            dimension_semantics=("parallel", "arbitrary")),
    )(use2, teach_oh, enc_states, enc_proj, emb_tab,
      wa, va, win, whh, bih, bhh, wout, bout, h_init)
    return hcat


def _proj_kernel(hcat_ref, wout_ref, bout_ref, logits_ref):
    logits_ref[...] = (
        jnp.dot(hcat_ref[...], wout_ref[...], preferred_element_type=jnp.float32)
        + bout_ref[...])


def _run_out_proj(hcat, wout, bout):
    n_steps, B, H2 = hcat.shape
    Vp = wout.shape[1]
    rows = n_steps * B
    flat = hcat.reshape(rows, H2)
    n_chunks = 4
    chunk = rows // n_chunks
    logits = pl.pallas_call(
        _proj_kernel,
        out_shape=jax.ShapeDtypeStruct((rows, Vp), jnp.float32),
        grid=(n_chunks,),
        in_specs=[
            pl.BlockSpec((chunk, H2), lambda i: (i, 0)),
            pl.BlockSpec((H2, Vp), lambda i: (0, 0)),
            pl.BlockSpec((1, Vp), lambda i: (0, 0)),
        ],
        out_specs=pl.BlockSpec((chunk, Vp), lambda i: (i, 0)),
        compiler_params=pltpu.CompilerParams(
            dimension_semantics=("arbitrary",)),
    )(flat, wout, bout)
    return logits.reshape(n_steps, B, Vp)


@partial(jax.jit, static_argnames=("vocab",))
def _forward(src_emb, trg_emb, enc_wih, enc_whh, enc_bih, enc_bhh,
             dec_wa, dec_ua, dec_va, dec_w_in, dec_whh, dec_bih, dec_bhh,
             dec_w_out, dec_bout, src, trg, use_teacher, *, vocab):
    max_len, batch = trg.shape
    Vp = dec_bout.shape[1]

    emb_src = jnp.take(src_emb, src, axis=0)
    enc_states, enc_proj, hidden = _run_encoder(
        emb_src, enc_wih, enc_whh, enc_bih, enc_bhh, dec_ua)

    teach_oh = jax.nn.one_hot(trg[:max_len - 1], Vp, dtype=jnp.float32)
    nxt = jnp.concatenate([use_teacher[1:], jnp.ones((1,), jnp.int32)])
    use2 = jnp.stack([use_teacher, nxt])
    hcat = _run_decoder(use2, teach_oh, enc_states, enc_proj, hidden,
                        trg_emb, dec_wa, dec_va, dec_w_in, dec_whh,
                        dec_bih, dec_bhh, dec_w_out, dec_bout, vocab=vocab)
    logits = _run_out_proj(hcat, dec_w_out, dec_bout)

    return jnp.concatenate(
        [jnp.zeros((1, batch, vocab), jnp.float32), logits[:, :, :vocab]], axis=0)


def kernel(src_emb, trg_emb, enc_wih, enc_whh, enc_bih, enc_bhh,
           dec_wa, dec_ua, dec_va, dec_w_in, dec_whh, dec_bih, dec_bhh,
           dec_w_out, dec_bout, src, trg, use_teacher):
    return _forward(src_emb, trg_emb, enc_wih, enc_whh, enc_bih, enc_bhh,
                    dec_wa, dec_ua, dec_va, dec_w_in, dec_whh, dec_bih, dec_bhh,
                    dec_w_out, dec_bout, src, trg, use_teacher, vocab=4096)


# speedup vs baseline: 1.0570x; 1.0570x over previous
from functools import partial

import jax
import jax.numpy as jnp
from jax.experimental import pallas as pl
from jax.experimental.pallas import tpu as pltpu

_NCORES = 1


def _enc_kernel(x_ref, wih_ref, whh_ref, bih_ref, bhh_ref, ua_ref,
                states_ref, projs_ref, hfinal_ref, h_scr):
    t = pl.program_id(1)
    Hp = h_scr.shape[1]

    @pl.when(t == 0)
    def _():
        h_scr[...] = jnp.zeros_like(h_scr)

    x = x_ref[0]
    h = h_scr[...]

    gx = jnp.dot(x, wih_ref[...], preferred_element_type=jnp.float32) + bih_ref[...]
    gh = jnp.dot(h, whh_ref[...], preferred_element_type=jnp.float32) + bhh_ref[...]

    r = jax.nn.sigmoid(gx[:, :Hp] + gh[:, :Hp])
    z = jax.nn.sigmoid(gx[:, Hp:2 * Hp] + gh[:, Hp:2 * Hp])
    n = jnp.tanh(gx[:, 2 * Hp:] + r * gh[:, 2 * Hp:])
    h_new = (1.0 - z) * n + z * h

    h_scr[...] = h_new
    states_ref[0] = h_new
    projs_ref[0] = jnp.dot(h_new, ua_ref[...], preferred_element_type=jnp.float32)

    @pl.when(t == pl.num_programs(1) - 1)
    def _():
        hfinal_ref[...] = h_new


def _run_encoder(emb_src, enc_wih, enc_whh, enc_bih, enc_bhh, dec_ua):
    T, B, Ep = emb_src.shape
    Hp = enc_whh.shape[0]
    Bblk = B // _NCORES
    states, projs, h_final = pl.pallas_call(
        _enc_kernel,
        out_shape=(jax.ShapeDtypeStruct((T, B, Hp), jnp.float32),
                   jax.ShapeDtypeStruct((T, B, Hp), jnp.float32),
                   jax.ShapeDtypeStruct((B, Hp), jnp.float32)),
        grid_spec=pltpu.PrefetchScalarGridSpec(
            num_scalar_prefetch=0,
            grid=(_NCORES, T),
            in_specs=[
                pl.BlockSpec((1, Bblk, Ep), lambda b, t: (t, b, 0)),
                pl.BlockSpec((Ep, 3 * Hp), lambda b, t: (0, 0)),
                pl.BlockSpec((Hp, 3 * Hp), lambda b, t: (0, 0)),
                pl.BlockSpec((1, 3 * Hp), lambda b, t: (0, 0)),
                pl.BlockSpec((1, 3 * Hp), lambda b, t: (0, 0)),
                pl.BlockSpec((Hp, Hp), lambda b, t: (0, 0)),
            ],
            out_specs=[
                pl.BlockSpec((1, Bblk, Hp), lambda b, t: (t, b, 0)),
                pl.BlockSpec((1, Bblk, Hp), lambda b, t: (t, b, 0)),
                pl.BlockSpec((Bblk, Hp), lambda b, t: (b, 0)),
            ],
            scratch_shapes=[pltpu.VMEM((Bblk, Hp), jnp.float32)],
        ),
        compiler_params=pltpu.CompilerParams(
            dimension_semantics=("parallel", "arbitrary")),
    )(emb_src, enc_wih, enc_whh, enc_bih, enc_bhh, dec_ua)
    return states, projs, h_final


def _dec_kernel(use_ref,
                teach_ref, enc_ref, projs_ref, emb_tab_ref,
                wa_ref, va_ref, win_ref, whh_ref, bih_ref, bhh_ref,
                wout_ref, bout_ref, hinit_ref,
                hcat_ref, h_scr, oh_scr, emb_scr, *, vocab):
    g = pl.program_id(1)
    B, Hp = h_scr.shape
    Vp = oh_scr.shape[1]
    nu = use_ref[1, g]

    @pl.when(g == 0)
    def _():
        h_scr[...] = hinit_ref[...]
        oh_scr[...] = jnp.zeros_like(oh_scr)

    h = h_scr[...]

    @pl.when(use_ref[0, g] > 0)
    def _():
        emb_scr[...] = teach_ref[0]

    @pl.when(use_ref[0, g] == 0)
    def _():
        emb_scr[...] = jnp.dot(oh_scr[...], emb_tab_ref[...],
                               preferred_element_type=jnp.float32)

    emb = emb_scr[...]

    proj_h = jnp.dot(h, wa_ref[...], preferred_element_type=jnp.float32)
    energy = jnp.tanh(projs_ref[...] + proj_h[None, :, :])
    scores = jnp.sum(energy * va_ref[...][None, :, :], axis=-1)
    scores = scores - jnp.max(scores, axis=0, keepdims=True)
    expo = jnp.exp(scores)
    alpha = expo * pl.reciprocal(jnp.sum(expo, axis=0, keepdims=True), approx=True)
    context = jnp.sum(alpha[:, :, None] * enc_ref[...], axis=0)

    xcat = jnp.concatenate([emb, context], axis=-1)
    gx = jnp.dot(xcat, win_ref[...], preferred_element_type=jnp.float32) + bih_ref[...]
    gh = jnp.dot(h, whh_ref[...], preferred_element_type=jnp.float32) + bhh_ref[...]
    r = jax.nn.sigmoid(gx[:, :Hp] + gh[:, :Hp])
    z = jax.nn.sigmoid(gx[:, Hp:2 * Hp] + gh[:, Hp:2 * Hp])
    n = jnp.tanh(gx[:, 2 * Hp:] + r * gh[:, 2 * Hp:])
    h_new = (1.0 - z) * n + z * h

    hcat = jnp.concatenate([h_new, context], axis=-1)
    hcat_ref[0] = hcat

    @pl.when(nu == 0)
    def _():
        logits = jnp.dot(hcat, wout_ref[...], preferred_element_type=jnp.float32) + bout_ref[...]
        v_iota = jax.lax.broadcasted_iota(jnp.int32, (B, Vp), 1).astype(jnp.float32)
        masked = jnp.where(v_iota < float(vocab), logits, -1e30)
        row_max = jnp.max(masked, axis=-1, keepdims=True)
        cand = jnp.where(masked == row_max, v_iota, float(Vp))
        first_idx = jnp.min(cand, axis=-1, keepdims=True)
        oh_scr[...] = (v_iota == first_idx).astype(jnp.float32)

    h_scr[...] = h_new


def _run_decoder(use2, teach_emb, enc_states, enc_proj, h_init,
                 emb_tab, wa, va, win, whh, bih, bhh, wout, bout, *, vocab):
    n_steps, B = teach_emb.shape[0], teach_emb.shape[1]
    T = enc_states.shape[0]
    Hp = h_init.shape[1]
    Ep = emb_tab.shape[1]
    Vp = emb_tab.shape[0]
    Bblk = B // _NCORES
    kern = partial(_dec_kernel, vocab=vocab)
    hcat = pl.pallas_call(
        kern,
        out_shape=jax.ShapeDtypeStruct((n_steps, B, 2 * Hp), jnp.float32),
        grid_spec=pltpu.PrefetchScalarGridSpec(
            num_scalar_prefetch=1,
            grid=(_NCORES, n_steps),
            in_specs=[
                pl.BlockSpec((1, Bblk, Ep), lambda b, g, u: (g, b, 0)),
                pl.BlockSpec((T, Bblk, Hp), lambda b, g, u: (0, b, 0)),
                pl.BlockSpec((T, Bblk, Hp), lambda b, g, u: (0, b, 0)),
                pl.BlockSpec((Vp, Ep), lambda b, g, u: (0, 0)),
                pl.BlockSpec((Hp, Hp), lambda b, g, u: (0, 0)),
                pl.BlockSpec((1, Hp), lambda b, g, u: (0, 0)),
                pl.BlockSpec((Ep + Hp, 3 * Hp), lambda b, g, u: (0, 0)),
                pl.BlockSpec((Hp, 3 * Hp), lambda b, g, u: (0, 0)),
                pl.BlockSpec((1, 3 * Hp), lambda b, g, u: (0, 0)),
                pl.BlockSpec((1, 3 * Hp), lambda b, g, u: (0, 0)),
                pl.BlockSpec((2 * Hp, Vp), lambda b, g, u: (0, 0)),
                pl.BlockSpec((1, Vp), lambda b, g, u: (0, 0)),
                pl.BlockSpec((Bblk, Hp), lambda b, g, u: (b, 0)),
            ],
            out_specs=pl.BlockSpec((1, Bblk, 2 * Hp), lambda b, g, u: (g, b, 0)),
            scratch_shapes=[pltpu.VMEM((Bblk, Hp), jnp.float32),
                            pltpu.VMEM((Bblk, Vp), jnp.float32),
                            pltpu.VMEM((Bblk, Ep), jnp.float32)],
        ),
        compiler_params=pltpu.CompilerParams(
            dimension_semantics=("parallel", "arbitrary")),
    )(use2, teach_emb, enc_states, enc_proj, emb_tab,
      wa, va, win, whh, bih, bhh, wout, bout, h_init)
    return hcat


def _proj_kernel(hcat_ref, wout_ref, bout_ref, logits_ref):
    logits_ref[...] = (
        jnp.dot(hcat_ref[...], wout_ref[...], preferred_element_type=jnp.float32)
        + bout_ref[...])


def _run_out_proj(hcat, wout, bout):
    n_steps, B, H2 = hcat.shape
    Vp = wout.shape[1]
    rows = n_steps * B
    flat = hcat.reshape(rows, H2)
    n_chunks = 4
    chunk = rows // n_chunks
    logits = pl.pallas_call(
        _proj_kernel,
        out_shape=jax.ShapeDtypeStruct((rows, Vp), jnp.float32),
        grid=(n_chunks,),
        in_specs=[
            pl.BlockSpec((chunk, H2), lambda i: (i, 0)),
            pl.BlockSpec((H2, Vp), lambda i: (0, 0)),
            pl.BlockSpec((1, Vp), lambda i: (0, 0)),
        ],
        out_specs=pl.BlockSpec((chunk, Vp), lambda i: (i, 0)),
        compiler_params=pltpu.CompilerParams(
            dimension_semantics=("arbitrary",)),
    )(flat, wout, bout)
    return logits.reshape(n_steps, B, Vp)


@partial(jax.jit, static_argnames=("vocab",))
def _forward(src_emb, trg_emb, enc_wih, enc_whh, enc_bih, enc_bhh,
             dec_wa, dec_ua, dec_va, dec_w_in, dec_whh, dec_bih, dec_bhh,
             dec_w_out, dec_bout, src, trg, use_teacher, *, vocab):
    max_len, batch = trg.shape
    Vp = dec_bout.shape[1]

    emb_src = jnp.take(src_emb, src, axis=0)
    enc_states, enc_proj, hidden = _run_encoder(
        emb_src, enc_wih, enc_whh, enc_bih, enc_bhh, dec_ua)

    teach_emb = jnp.take(trg_emb, trg[:max_len - 1], axis=0)
    nxt = jnp.concatenate([use_teacher[1:], jnp.ones((1,), jnp.int32)])
    use2 = jnp.stack([use_teacher, nxt])
    hcat = _run_decoder(use2, teach_emb, enc_states, enc_proj, hidden,
                        trg_emb, dec_wa, dec_va, dec_w_in, dec_whh,
                        dec_bih, dec_bhh, dec_w_out, dec_bout, vocab=vocab)
    logits = _run_out_proj(hcat, dec_w_out, dec_bout)

    return jnp.concatenate(
        [jnp.zeros((1, batch, vocab), jnp.float32), logits[:, :, :vocab]], axis=0)


def kernel(src_emb, trg_emb, enc_wih, enc_whh, enc_bih, enc_bhh,
           dec_wa, dec_ua, dec_va, dec_w_in, dec_whh, dec_bih, dec_bhh,
           dec_w_out, dec_bout, src, trg, use_teacher):
    return _forward(src_emb, trg_emb, enc_wih, enc_whh, enc_bih, enc_bhh,
                    dec_wa, dec_ua, dec_va, dec_w_in, dec_whh, dec_bih, dec_bhh,
                    dec_w_out, dec_bout, src, trg, use_teacher, vocab=4096)


# speedup vs baseline: 1.0742x; 1.0163x over previous
from functools import partial

import jax
import jax.numpy as jnp
from jax.experimental import pallas as pl
from jax.experimental.pallas import tpu as pltpu

_NCORES = 1


def _enc_kernel(x_ref, wih_ref, whh_ref, bih_ref, bhh_ref, ua_ref,
                states_ref, projs_ref, hfinal_ref, h_scr):
    t = pl.program_id(1)
    Hp = h_scr.shape[1]

    @pl.when(t == 0)
    def _():
        h_scr[...] = jnp.zeros_like(h_scr)

    x = x_ref[0]
    h = h_scr[...]

    gx = jnp.dot(x, wih_ref[...], preferred_element_type=jnp.float32) + bih_ref[...]
    gh = jnp.dot(h, whh_ref[...], preferred_element_type=jnp.float32) + bhh_ref[...]

    r = jax.nn.sigmoid(gx[:, :Hp] + gh[:, :Hp])
    z = jax.nn.sigmoid(gx[:, Hp:2 * Hp] + gh[:, Hp:2 * Hp])
    n = jnp.tanh(gx[:, 2 * Hp:] + r * gh[:, 2 * Hp:])
    h_new = (1.0 - z) * n + z * h

    h_scr[...] = h_new
    states_ref[0] = h_new
    projs_ref[0] = jnp.dot(h_new, ua_ref[...], preferred_element_type=jnp.float32)

    @pl.when(t == pl.num_programs(1) - 1)
    def _():
        hfinal_ref[...] = h_new


def _run_encoder(emb_src, enc_wih, enc_whh, enc_bih, enc_bhh, dec_ua):
    T, B, Ep = emb_src.shape
    Hp = enc_whh.shape[0]
    Bblk = B // _NCORES
    states, projs, h_final = pl.pallas_call(
        _enc_kernel,
        out_shape=(jax.ShapeDtypeStruct((T, B, Hp), jnp.float32),
                   jax.ShapeDtypeStruct((T, B, Hp), jnp.float32),
                   jax.ShapeDtypeStruct((B, Hp), jnp.float32)),
        grid_spec=pltpu.PrefetchScalarGridSpec(
            num_scalar_prefetch=0,
            grid=(_NCORES, T),
            in_specs=[
                pl.BlockSpec((1, Bblk, Ep), lambda b, t: (t, b, 0)),
                pl.BlockSpec((Ep, 3 * Hp), lambda b, t: (0, 0)),
                pl.BlockSpec((Hp, 3 * Hp), lambda b, t: (0, 0)),
                pl.BlockSpec((1, 3 * Hp), lambda b, t: (0, 0)),
                pl.BlockSpec((1, 3 * Hp), lambda b, t: (0, 0)),
                pl.BlockSpec((Hp, Hp), lambda b, t: (0, 0)),
            ],
            out_specs=[
                pl.BlockSpec((1, Bblk, Hp), lambda b, t: (t, b, 0)),
                pl.BlockSpec((1, Bblk, Hp), lambda b, t: (t, b, 0)),
                pl.BlockSpec((Bblk, Hp), lambda b, t: (b, 0)),
            ],
            scratch_shapes=[pltpu.VMEM((Bblk, Hp), jnp.float32)],
        ),
        compiler_params=pltpu.CompilerParams(
            dimension_semantics=("parallel", "arbitrary")),
    )(emb_src, enc_wih, enc_whh, enc_bih, enc_bhh, dec_ua)
    return states, projs, h_final


def _dec_kernel(use_ref,
                teach_ref, enc_ref, projs_ref, emb_tab_ref,
                wa_ref, va_ref, win_ref, whh_ref, bih_ref, bhh_ref,
                wout_ref, bout_ref, hinit_ref,
                logits_ref, h_scr, oh_scr, emb_scr, *, vocab):
    g = pl.program_id(1)
    B, Hp = h_scr.shape
    Vp = oh_scr.shape[1]
    nu = use_ref[1, g]

    @pl.when(g == 0)
    def _():
        h_scr[...] = hinit_ref[...]
        oh_scr[...] = jnp.zeros_like(oh_scr)

    h = h_scr[...]

    @pl.when(use_ref[0, g] > 0)
    def _():
        emb_scr[...] = teach_ref[0]

    @pl.when(use_ref[0, g] == 0)
    def _():
        emb_scr[...] = jnp.dot(oh_scr[...], emb_tab_ref[...],
                               preferred_element_type=jnp.float32)

    emb = emb_scr[...]

    proj_h = jnp.dot(h, wa_ref[...], preferred_element_type=jnp.float32)
    energy = jnp.tanh(projs_ref[...] + proj_h[None, :, :])
    scores = jnp.sum(energy * va_ref[...][None, :, :], axis=-1)
    scores = scores - jnp.max(scores, axis=0, keepdims=True)
    expo = jnp.exp(scores)
    alpha = expo * pl.reciprocal(jnp.sum(expo, axis=0, keepdims=True), approx=True)
    context = jnp.sum(alpha[:, :, None] * enc_ref[...], axis=0)

    xcat = jnp.concatenate([emb, context], axis=-1)
    gx = jnp.dot(xcat, win_ref[...], preferred_element_type=jnp.float32) + bih_ref[...]
    gh = jnp.dot(h, whh_ref[...], preferred_element_type=jnp.float32) + bhh_ref[...]
    r = jax.nn.sigmoid(gx[:, :Hp] + gh[:, :Hp])
    z = jax.nn.sigmoid(gx[:, Hp:2 * Hp] + gh[:, Hp:2 * Hp])
    n = jnp.tanh(gx[:, 2 * Hp:] + r * gh[:, 2 * Hp:])
    h_new = (1.0 - z) * n + z * h

    hcat = jnp.concatenate([h_new, context], axis=-1)
    logits = jnp.dot(hcat, wout_ref[...], preferred_element_type=jnp.float32) + bout_ref[...]
    logits_ref[0] = logits

    @pl.when(nu == 0)
    def _():
        v_iota = jax.lax.broadcasted_iota(jnp.int32, (B, Vp), 1).astype(jnp.float32)
        masked = jnp.where(v_iota < float(vocab), logits, -1e30)
        row_max = jnp.max(masked, axis=-1, keepdims=True)
        cand = jnp.where(masked == row_max, v_iota, float(Vp))
        first_idx = jnp.min(cand, axis=-1, keepdims=True)
        oh_scr[...] = (v_iota == first_idx).astype(jnp.float32)

    h_scr[...] = h_new


def _run_decoder(use2, teach_emb, enc_states, enc_proj, h_init,
                 emb_tab, wa, va, win, whh, bih, bhh, wout, bout, *, vocab):
    n_steps, B = teach_emb.shape[0], teach_emb.shape[1]
    T = enc_states.shape[0]
    Hp = h_init.shape[1]
    Ep = emb_tab.shape[1]
    Vp = emb_tab.shape[0]
    Bblk = B // _NCORES
    kern = partial(_dec_kernel, vocab=vocab)
    logits = pl.pallas_call(
        kern,
        out_shape=jax.ShapeDtypeStruct((n_steps, B, Vp), jnp.float32),
        grid_spec=pltpu.PrefetchScalarGridSpec(
            num_scalar_prefetch=1,
            grid=(_NCORES, n_steps),
            in_specs=[
                pl.BlockSpec((1, Bblk, Ep), lambda b, g, u: (g, b, 0)),
                pl.BlockSpec((T, Bblk, Hp), lambda b, g, u: (0, b, 0)),
                pl.BlockSpec((T, Bblk, Hp), lambda b, g, u: (0, b, 0)),
                pl.BlockSpec((Vp, Ep), lambda b, g, u: (0, 0)),
                pl.BlockSpec((Hp, Hp), lambda b, g, u: (0, 0)),
                pl.BlockSpec((1, Hp), lambda b, g, u: (0, 0)),
                pl.BlockSpec((Ep + Hp, 3 * Hp), lambda b, g, u: (0, 0)),
                pl.BlockSpec((Hp, 3 * Hp), lambda b, g, u: (0, 0)),
                pl.BlockSpec((1, 3 * Hp), lambda b, g, u: (0, 0)),
                pl.BlockSpec((1, 3 * Hp), lambda b, g, u: (0, 0)),
                pl.BlockSpec((2 * Hp, Vp), lambda b, g, u: (0, 0)),
                pl.BlockSpec((1, Vp), lambda b, g, u: (0, 0)),
                pl.BlockSpec((Bblk, Hp), lambda b, g, u: (b, 0)),
            ],
            out_specs=pl.BlockSpec((1, Bblk, Vp), lambda b, g, u: (g, b, 0)),
            scratch_shapes=[pltpu.VMEM((Bblk, Hp), jnp.float32),
                            pltpu.VMEM((Bblk, Vp), jnp.float32),
                            pltpu.VMEM((Bblk, Ep), jnp.float32)],
        ),
        compiler_params=pltpu.CompilerParams(
            dimension_semantics=("parallel", "arbitrary")),
    )(use2, teach_emb, enc_states, enc_proj, emb_tab,
      wa, va, win, whh, bih, bhh, wout, bout, h_init)
    return logits


@partial(jax.jit, static_argnames=("vocab",))
def _forward(src_emb, trg_emb, enc_wih, enc_whh, enc_bih, enc_bhh,
             dec_wa, dec_ua, dec_va, dec_w_in, dec_whh, dec_bih, dec_bhh,
             dec_w_out, dec_bout, src, trg, use_teacher, *, vocab):
    max_len, batch = trg.shape
    Vp = dec_bout.shape[1]

    emb_src = jnp.take(src_emb, src, axis=0)
    enc_states, enc_proj, hidden = _run_encoder(
        emb_src, enc_wih, enc_whh, enc_bih, enc_bhh, dec_ua)

    teach_emb = jnp.take(trg_emb, trg[:max_len - 1], axis=0)
    nxt = jnp.concatenate([use_teacher[1:], jnp.ones((1,), jnp.int32)])
    use2 = jnp.stack([use_teacher, nxt])
    logits = _run_decoder(use2, teach_emb, enc_states, enc_proj, hidden,
                          trg_emb, dec_wa, dec_va, dec_w_in, dec_whh,
                          dec_bih, dec_bhh, dec_w_out, dec_bout, vocab=vocab)

    return jnp.concatenate(
        [jnp.zeros((1, batch, vocab), jnp.float32), logits[:, :, :vocab]], axis=0)


def kernel(src_emb, trg_emb, enc_wih, enc_whh, enc_bih, enc_bhh,
           dec_wa, dec_ua, dec_va, dec_w_in, dec_whh, dec_bih, dec_bhh,
           dec_w_out, dec_bout, src, trg, use_teacher):
    return _forward(src_emb, trg_emb, enc_wih, enc_whh, enc_bih, enc_bhh,
                    dec_wa, dec_ua, dec_va, dec_w_in, dec_whh, dec_bih, dec_bhh,
                    dec_w_out, dec_bout, src, trg, use_teacher, vocab=4096)
